# baseline (speedup 1.0000x reference)
import functools

import jax
import jax.numpy as jnp
from jax import lax
from jax.experimental import pallas as pl
from jax.experimental.pallas import tpu as pltpu
from jax.experimental.pallas import tpu_sc as plsc

DIM = 64
NUM_CORES = 2
NUM_SUBCORES = 16
NW = NUM_CORES * NUM_SUBCORES
TOTAL = 4096 * 200
PER_W = TOTAL // NW
CHUNK = 800
NCHUNK = PER_W // CHUNK

_mesh = plsc.VectorSubcoreMesh(core_axis_name="c", subcore_axis_name="s")


@functools.partial(
    pl.kernel,
    out_type=jax.ShapeDtypeStruct((TOTAL, DIM), jnp.float32),
    mesh=_mesh,
    scratch_types=[
        pltpu.VMEM((CHUNK,), jnp.int32),
        pltpu.VMEM((CHUNK, DIM), jnp.float32),
        pltpu.SemaphoreType.DMA,
    ],
    compiler_params=pltpu.CompilerParams(use_tc_tiling_on_sc=False),
)
def _gather_kernel(idx_hbm, table_hbm, out_hbm, idx_v, rows_v, sem):
    wid = lax.axis_index("s") * NUM_CORES + lax.axis_index("c")
    base = wid * PER_W

    def body(j, carry):
        off = base + j * CHUNK
        pltpu.sync_copy(idx_hbm.at[pl.ds(off, CHUNK)], idx_v)
        pltpu.async_copy(table_hbm.at[idx_v], rows_v, sem).wait()
        pltpu.sync_copy(rows_v, out_hbm.at[pl.ds(off, CHUNK)])
        return carry

    lax.fori_loop(0, NCHUNK, body, 0)


def kernel(input_ids, table):
    flat = input_ids.reshape(-1).astype(jnp.int32)
    out = _gather_kernel(flat, table)
    return out.reshape(input_ids.shape + (DIM,))


# speedup vs baseline: 1.0236x; 1.0236x over previous
import functools

import jax
import jax.numpy as jnp
from jax import lax
from jax.experimental import pallas as pl
from jax.experimental.pallas import tpu as pltpu
from jax.experimental.pallas import tpu_sc as plsc

DIM = 64
NUM_CORES = 2
NUM_SUBCORES = 16
NW = NUM_CORES * NUM_SUBCORES
TOTAL = 4096 * 200
PER_W = TOTAL // NW
CHUNK = 640
NCHUNK = PER_W // CHUNK
NPAIR = NCHUNK // 2

_mesh = plsc.VectorSubcoreMesh(core_axis_name="c", subcore_axis_name="s")


@functools.partial(
    pl.kernel,
    out_type=jax.ShapeDtypeStruct((TOTAL, DIM), jnp.float32),
    mesh=_mesh,
    scratch_types=[
        pltpu.VMEM((NCHUNK, CHUNK), jnp.int32),
        pltpu.VMEM((2, CHUNK, DIM), jnp.float32),
        pltpu.SemaphoreType.DMA((2,)),
        pltpu.SemaphoreType.DMA((2,)),
    ],
    compiler_params=pltpu.CompilerParams(use_tc_tiling_on_sc=False),
)
def _gather_kernel(idx_hbm, table_hbm, out_hbm, idx_v, rows_v, gsem, ssem):
    wid = lax.axis_index("s") * NUM_CORES + lax.axis_index("c")
    base = wid * PER_W

    pltpu.sync_copy(idx_hbm.at[wid], idx_v)

    def gather(j, b):
        return pltpu.async_copy(table_hbm.at[idx_v.at[j]], rows_v.at[b],
                                gsem.at[b])

    def store(j, b):
        return pltpu.async_copy(
            rows_v.at[b], out_hbm.at[pl.ds(base + j * CHUNK, CHUNK)],
            ssem.at[b])

    def wait_g(b):
        pltpu.make_async_copy(table_hbm.at[idx_v.at[0]], rows_v.at[b],
                              gsem.at[b]).wait()

    def wait_s(b):
        pltpu.make_async_copy(rows_v.at[b],
                              out_hbm.at[pl.ds(base, CHUNK)],
                              ssem.at[b]).wait()

    gather(0, 0)
    gather(1, 1)

    def body(g, carry):
        j0 = 2 * g
        wait_g(0)
        store(j0, 0)
        wait_g(1)
        store(j0 + 1, 1)

        @pl.when(g < NPAIR - 1)
        def _():
            wait_s(0)
            gather(j0 + 2, 0)
            wait_s(1)
            gather(j0 + 3, 1)

        return carry

    lax.fori_loop(0, NPAIR, body, 0)
    wait_s(0)
    wait_s(1)


def kernel(input_ids, table):
    flat = input_ids.reshape(NW, NCHUNK, CHUNK).astype(jnp.int32)
    out = _gather_kernel(flat, table)
    return out.reshape(input_ids.shape + (DIM,))


# speedup vs baseline: 1.2477x; 1.2189x over previous
import functools

import jax
import jax.numpy as jnp
from jax import lax
from jax.experimental import pallas as pl
from jax.experimental.pallas import tpu as pltpu
from jax.experimental.pallas import tpu_sc as plsc

DIM = 64
PDIM = 128
NUM_CORES = 2
NUM_SUBCORES = 16
NW = NUM_CORES * NUM_SUBCORES
TOTAL = 4096 * 200
PER_W = TOTAL // NW
CHUNK = 256
NCHUNK = PER_W // CHUNK
NPAIR = NCHUNK // 2

_mesh = plsc.VectorSubcoreMesh(core_axis_name="c", subcore_axis_name="s")


@functools.partial(
    pl.kernel,
    out_type=jax.ShapeDtypeStruct((TOTAL, PDIM), jnp.float32),
    mesh=_mesh,
    scratch_types=[
        pltpu.VMEM((PER_W,), jnp.int32),
        pltpu.VMEM((2, CHUNK, PDIM), jnp.float32),
        pltpu.SemaphoreType.DMA((2,)),
        pltpu.SemaphoreType.DMA((2,)),
    ],
    compiler_params=pltpu.CompilerParams(use_tc_tiling_on_sc=True),
)
def _gather_kernel(idx_hbm, table_hbm, out_hbm, idx_v, rows_v, gsem, ssem):
    wid = lax.axis_index("s") * NUM_CORES + lax.axis_index("c")
    base = wid * PER_W

    pltpu.sync_copy(idx_hbm.at[pl.ds(base, PER_W)], idx_v)

    def gather(j, b):
        pltpu.async_copy(
            table_hbm.at[idx_v.at[pl.ds(j * CHUNK, CHUNK)]], rows_v.at[b],
            gsem.at[b])

    def store(j, b):
        pltpu.async_copy(
            rows_v.at[b], out_hbm.at[pl.ds(base + j * CHUNK, CHUNK)],
            ssem.at[b])

    def wait_g(b):
        pltpu.make_async_copy(
            table_hbm.at[idx_v.at[pl.ds(0, CHUNK)]], rows_v.at[b],
            gsem.at[b]).wait()

    def wait_s(b):
        pltpu.make_async_copy(
            rows_v.at[b], out_hbm.at[pl.ds(base, CHUNK)],
            ssem.at[b]).wait()

    gather(0, 0)
    gather(1, 1)

    def body(g, carry):
        j0 = 2 * g
        wait_g(0)
        store(j0, 0)
        wait_g(1)
        store(j0 + 1, 1)

        @pl.when(g < NPAIR - 1)
        def _():
            wait_s(0)
            gather(j0 + 2, 0)
            wait_s(1)
            gather(j0 + 3, 1)

        return carry

    lax.fori_loop(0, NPAIR, body, 0)
    wait_s(0)
    wait_s(1)


def kernel(input_ids, table):
    flat = input_ids.reshape(-1).astype(jnp.int32)
    table_p = jnp.pad(table, ((0, 0), (0, PDIM - DIM)))
    out = _gather_kernel(flat, table_p)
    return out[:, :DIM].reshape(input_ids.shape + (DIM,))
